# baseline (speedup 1.0000x reference)
import functools

import jax
import jax.numpy as jnp
from jax import lax
from jax.experimental import pallas as pl
from jax.experimental.pallas import tpu as pltpu
from jax.experimental.pallas import tpu_sc as plsc

S = 2048
D = 64
H = 12
CB = 2 * D


def _roll_up(a, j):
    return jnp.concatenate([a[j:], a[:j]], axis=0)


def _roll_dn(a, j):
    return jnp.concatenate([a[-j:], a[:-j]], axis=0)


def _bitonic_stages(n):
    kk = 2
    while kk <= n:
        j = kk // 2
        while j >= 1:
            yield kk, j
            j //= 2
        kk *= 2


def _dyn_partner(a, j, n, m_low):
    up = pltpu.roll(a, n - j, 0)
    dn = pltpu.roll(a, j, 0)
    return jnp.where(m_low, up, dn)


def _tc_sort_body(qk_ref, out_ref):
    n = qk_ref.shape[0]
    c = qk_ref.shape[1] // 2
    iota2 = lax.broadcasted_iota(jnp.int32, (n, 2 * c), 0)
    x0 = qk_ref[...]

    def pair_stage(t, carry):
        x, ix, kb = carry
        kk = jnp.left_shift(jnp.int32(1), kb)
        j = jnp.left_shift(jnp.int32(1), kb - 1 - t)
        m_low = (iota2 & j) == 0
        asc = (iota2 & kk) == 0
        want_min = m_low == asc
        xp = _dyn_partner(x, j, n, m_low)
        ixp = _dyn_partner(ix, j, n, m_low)
        lt = (x < xp) | ((x == xp) & (ix < ixp))
        tp = want_min != lt
        return jnp.where(tp, xp, x), jnp.where(tp, ixp, ix), kb

    def pair_outer(kb, carry):
        x, ix = carry
        x, ix, _ = lax.fori_loop(0, kb, pair_stage, (x, ix, kb))
        return x, ix

    x, ix = lax.fori_loop(1, 12, pair_outer, (x0, iota2))

    qs = x[:, :c]
    ks = x[:, c:]
    val = jnp.exp(-(qs - ks) * (qs - ks)) * jnp.float32(1.0 / D)
    iota1 = lax.broadcasted_iota(jnp.int32, (n, c), 0)

    def inv_stage(t, carry):
        kc, v, kb = carry
        kk = jnp.left_shift(jnp.int32(1), kb)
        j = jnp.left_shift(jnp.int32(1), kb - 1 - t)
        m_low1 = (iota1 & j) == 0
        asc1 = (iota1 & kk) == 0
        want_min1 = m_low1 == asc1
        m_low = (iota2 & j) == 0
        kcp = _dyn_partner(kc, j, n, m_low)
        vp = _dyn_partner(v, j, n, m_low1)
        lt = kc[:, :c] < kcp[:, :c]
        tp = want_min1 != lt
        key_new = jnp.where(tp, kcp[:, :c], kc[:, :c])
        col_new = jnp.where(tp, kcp[:, c:], kc[:, c:])
        return jnp.concatenate([key_new, col_new], axis=1), jnp.where(tp, vp, v), kb

    def inv_outer(kb, carry):
        kc, v = carry
        kc, v, _ = lax.fori_loop(0, kb, inv_stage, (kc, v, kb))
        return kc, v

    kc, v = lax.fori_loop(1, 12, inv_outer, (ix, val))
    out_ref[...] = jnp.concatenate(
        [kc[:, c:], lax.bitcast_convert_type(v, jnp.int32)], axis=1)


def _tc_sort(qk):
    return pl.pallas_call(
        _tc_sort_body,
        grid=(H,),
        in_specs=[pl.BlockSpec((S, CB), lambda i: (0, i))],
        out_specs=pl.BlockSpec((S, CB), lambda i: (0, i)),
        out_shape=jax.ShapeDtypeStruct((S, H * CB), jnp.int32),
    )(qk)


R = 32
SPAN = 128
NW = 32
SPANS = H * (S // SPAN)
PER_TILE = SPANS // NW
MW = S // 4

def _sc_scatter_body(col_hbm, val_hbm, mask_hbm, out_hbm, colb, valb, maskb, outb):
    cid = lax.axis_index("c")
    sid = lax.axis_index("s")
    wid = sid * 2 + cid
    lanes = lax.iota(jnp.int32, 16)

    def span_body(t, carry):
        span = wid + NW * t
        n = span // (S // SPAN)
        rs = (span % (S // SPAN)) * SPAN
        pltpu.sync_copy(col_hbm.at[pl.ds(span * (D * SPAN), D * SPAN)], colb)
        pltpu.sync_copy(val_hbm.at[pl.ds(span * (D * SPAN), D * SPAN)], valb)

        def sub_body(sub, carry2):
            r0 = rs + sub * R
            pltpu.sync_copy(mask_hbm.at[pl.ds((n * S + r0) * MW, R * MW)], maskb)

            def zcol(cc, carry3):
                outb[pl.ds(cc * 16, 16)] = jnp.zeros((16,), jnp.float32)
                return carry3
            lax.fori_loop(0, R * S // 16, zcol, 0)

            def feat_body(di, carry3):
                def grp(g, carry4):
                    e0 = di * SPAN + sub * R + g * 16
                    colv = colb[pl.ds(e0, 16)]
                    valv = valb[pl.ds(e0, 16)]
                    rloc = g * 16 + lanes
                    w = plsc.load_gather(
                        maskb, [rloc * MW + lax.shift_right_logical(colv, 2)])
                    byte = lax.shift_right_logical(w, (colv & 3) * 8) & 0xFF
                    keep = byte == 0
                    plsc.addupdate_scatter(outb, [rloc * S + colv], valv, mask=keep)
                    return carry4
                return lax.fori_loop(0, R // 16, grp, carry3)
            lax.fori_loop(0, D, feat_body, 0)

            pltpu.sync_copy(outb, out_hbm.at[pl.ds((n * S + r0) * S, R * S)])
            return carry2
        lax.fori_loop(0, SPAN // R, sub_body, 0)
        return carry
    lax.fori_loop(0, PER_TILE, span_body, 0)


@functools.lru_cache(maxsize=1)
def _get_sc_scatter():
    mesh = plsc.VectorSubcoreMesh(core_axis_name="c", subcore_axis_name="s")
    return functools.partial(
        pl.kernel,
        mesh=mesh,
        compiler_params=pltpu.CompilerParams(needs_layout_passes=False),
        out_type=jax.ShapeDtypeStruct((H * S * S,), jnp.float32),
        scratch_types=[
            pltpu.VMEM((D * SPAN,), jnp.int32),
            pltpu.VMEM((D * SPAN,), jnp.float32),
            pltpu.VMEM((R * MW,), jnp.int32),
            pltpu.VMEM((R * S,), jnp.float32),
        ],
    )(_sc_scatter_body)


def kernel(q, k, attn_mask):
    mask_shape = attn_mask.shape
    q2 = q.reshape(H, S, D).transpose(1, 0, 2)
    k2 = k.reshape(H, S, D).transpose(1, 0, 2)
    qk = jnp.concatenate([q2, k2], axis=2).reshape(S, H * CB)
    packed = _tc_sort(qk).reshape(S, H, CB)
    col = packed[:, :, :D].reshape(S, H * D)
    val = lax.bitcast_convert_type(packed[:, :, D:], jnp.float32).reshape(S, H * D)
    col_t = col.reshape(S // SPAN, SPAN, H, D).transpose(2, 0, 3, 1).reshape(-1)
    val_t = val.reshape(S // SPAN, SPAN, H, D).transpose(2, 0, 3, 1).reshape(-1)
    m8 = attn_mask.reshape(H, S, S).view(jnp.uint8)
    mw = lax.bitcast_convert_type(m8.reshape(H * S * MW, 4), jnp.int32)
    out = _get_sc_scatter()(col_t, val_t, mw)
    return out.reshape(mask_shape)


# speedup vs baseline: 1.0739x; 1.0739x over previous
import functools

import jax
import jax.numpy as jnp
from jax import lax
from jax.experimental import pallas as pl
from jax.experimental.pallas import tpu as pltpu
from jax.experimental.pallas import tpu_sc as plsc

S = 2048
D = 64
H = 12
CB = 2 * D


def _roll_up(a, j):
    return jnp.concatenate([a[j:], a[:j]], axis=0)


def _roll_dn(a, j):
    return jnp.concatenate([a[-j:], a[:-j]], axis=0)


def _bitonic_stages(n):
    kk = 2
    while kk <= n:
        j = kk // 2
        while j >= 1:
            yield kk, j
            j //= 2
        kk *= 2


def _tc_sort_body(qk_ref, out_ref):
    n = qk_ref.shape[0]
    c = qk_ref.shape[1] // 2
    nb = n.bit_length() - 1
    iota2 = lax.broadcasted_iota(jnp.int32, (n, 2 * c), 0)
    x0 = qk_ref[...]

    def pair_stage(t, carry):
        x, ix, kb = carry
        kk = jnp.left_shift(jnp.int32(1), kb)
        asc = (iota2 & kk) == 0

        def mk(bs):
            j = 1 << bs

            def br(x, ix, asc):
                m_low = (iota2 & j) == 0
                want_min = m_low == asc
                xp = jnp.where(m_low, _roll_up(x, j), _roll_dn(x, j))
                ixp = jnp.where(m_low, _roll_up(ix, j), _roll_dn(ix, j))
                lt = (x < xp) | ((x == xp) & (ix < ixp))
                tp = want_min != lt
                return jnp.where(tp, xp, x), jnp.where(tp, ixp, ix)
            return br

        x, ix = lax.switch(kb - 1 - t, [mk(bs) for bs in range(nb)], x, ix, asc)
        return x, ix, kb

    def pair_outer(kb, carry):
        x, ix = carry
        x, ix, _ = lax.fori_loop(0, kb, pair_stage, (x, ix, kb))
        return x, ix

    x, ix = lax.fori_loop(1, nb + 1, pair_outer, (x0, iota2))

    qs = x[:, :c]
    ks = x[:, c:]
    val = jnp.exp(-(qs - ks) * (qs - ks)) * jnp.float32(1.0 / D)
    iota1 = lax.broadcasted_iota(jnp.int32, (n, c), 0)

    def inv_stage(t, carry):
        kc, v, kb = carry
        kk = jnp.left_shift(jnp.int32(1), kb)
        asc1 = (iota1 & kk) == 0

        def mk(bs):
            j = 1 << bs

            def br(kc, v, asc1):
                m_low1 = (iota1 & j) == 0
                want_min1 = m_low1 == asc1
                m_low = (iota2 & j) == 0
                kcp = jnp.where(m_low, _roll_up(kc, j), _roll_dn(kc, j))
                vp = jnp.where(m_low1, _roll_up(v, j), _roll_dn(v, j))
                lt = kc[:, :c] < kcp[:, :c]
                tp = want_min1 != lt
                key_new = jnp.where(tp, kcp[:, :c], kc[:, :c])
                col_new = jnp.where(tp, kcp[:, c:], kc[:, c:])
                return (jnp.concatenate([key_new, col_new], axis=1),
                        jnp.where(tp, vp, v))
            return br

        kc, v = lax.switch(kb - 1 - t, [mk(bs) for bs in range(nb)], kc, v, asc1)
        return kc, v, kb

    def inv_outer(kb, carry):
        kc, v = carry
        kc, v, _ = lax.fori_loop(0, kb, inv_stage, (kc, v, kb))
        return kc, v

    kc, v = lax.fori_loop(1, nb + 1, inv_outer, (ix, val))
    out_ref[...] = jnp.concatenate(
        [kc[:, c:], lax.bitcast_convert_type(v, jnp.int32)], axis=1)


def _tc_sort(qk):
    return pl.pallas_call(
        _tc_sort_body,
        grid=(H,),
        in_specs=[pl.BlockSpec((S, CB), lambda i: (0, i))],
        out_specs=pl.BlockSpec((S, CB), lambda i: (0, i)),
        out_shape=jax.ShapeDtypeStruct((S, H * CB), jnp.int32),
    )(qk)


R = 32
SPAN = 128
NW = 32
SPANS = H * (S // SPAN)
PER_TILE = SPANS // NW
MW = S // 4

def _sc_scatter_body(col_hbm, val_hbm, mask_hbm, out_hbm, colb, valb, maskb, outb):
    cid = lax.axis_index("c")
    sid = lax.axis_index("s")
    wid = sid * 2 + cid
    lanes = lax.iota(jnp.int32, 16)

    def span_body(t, carry):
        span = wid + NW * t
        n = span // (S // SPAN)
        rs = (span % (S // SPAN)) * SPAN
        pltpu.sync_copy(col_hbm.at[pl.ds(span * (D * SPAN), D * SPAN)], colb)
        pltpu.sync_copy(val_hbm.at[pl.ds(span * (D * SPAN), D * SPAN)], valb)

        def sub_body(sub, carry2):
            r0 = rs + sub * R
            pltpu.sync_copy(mask_hbm.at[pl.ds((n * S + r0) * MW, R * MW)], maskb)

            def zcol(cc, carry3):
                outb[pl.ds(cc * 16, 16)] = jnp.zeros((16,), jnp.float32)
                return carry3
            lax.fori_loop(0, R * S // 16, zcol, 0)

            def feat_body(di, carry3):
                def grp(g, carry4):
                    e0 = di * SPAN + sub * R + g * 16
                    colv = colb[pl.ds(e0, 16)]
                    valv = valb[pl.ds(e0, 16)]
                    rloc = g * 16 + lanes
                    w = plsc.load_gather(
                        maskb, [rloc * MW + lax.shift_right_logical(colv, 2)])
                    byte = lax.shift_right_logical(w, (colv & 3) * 8) & 0xFF
                    keep = byte == 0
                    plsc.addupdate_scatter(outb, [rloc * S + colv], valv, mask=keep)
                    return carry4
                return lax.fori_loop(0, R // 16, grp, carry3)
            lax.fori_loop(0, D, feat_body, 0)

            pltpu.sync_copy(outb, out_hbm.at[pl.ds((n * S + r0) * S, R * S)])
            return carry2
        lax.fori_loop(0, SPAN // R, sub_body, 0)
        return carry
    lax.fori_loop(0, PER_TILE, span_body, 0)


@functools.lru_cache(maxsize=1)
def _get_sc_scatter():
    mesh = plsc.VectorSubcoreMesh(core_axis_name="c", subcore_axis_name="s")
    return functools.partial(
        pl.kernel,
        mesh=mesh,
        compiler_params=pltpu.CompilerParams(needs_layout_passes=False),
        out_type=jax.ShapeDtypeStruct((H * S * S,), jnp.float32),
        scratch_types=[
            pltpu.VMEM((D * SPAN,), jnp.int32),
            pltpu.VMEM((D * SPAN,), jnp.float32),
            pltpu.VMEM((R * MW,), jnp.int32),
            pltpu.VMEM((R * S,), jnp.float32),
        ],
    )(_sc_scatter_body)


def kernel(q, k, attn_mask):
    mask_shape = attn_mask.shape
    q2 = q.reshape(H, S, D).transpose(1, 0, 2)
    k2 = k.reshape(H, S, D).transpose(1, 0, 2)
    qk = jnp.concatenate([q2, k2], axis=2).reshape(S, H * CB)
    packed = _tc_sort(qk).reshape(S, H, CB)
    col = packed[:, :, :D].reshape(S, H * D)
    val = lax.bitcast_convert_type(packed[:, :, D:], jnp.float32).reshape(S, H * D)
    col_t = col.reshape(S // SPAN, SPAN, H, D).transpose(2, 0, 3, 1).reshape(-1)
    val_t = val.reshape(S // SPAN, SPAN, H, D).transpose(2, 0, 3, 1).reshape(-1)
    m8 = attn_mask.reshape(H, S, S).view(jnp.uint8)
    mw = lax.bitcast_convert_type(m8.reshape(H * S * MW, 4), jnp.int32)
    out = _get_sc_scatter()(col_t, val_t, mw)
    return out.reshape(mask_shape)
